# baseline (speedup 1.0000x reference)
import jax
import jax.numpy as jnp
from jax.experimental import pallas as pl

N = 10000
H1, C1 = 8, 64
H2, C2 = 8, 16


def _matmul(x, w, b):
    m, k = x.shape
    _, n = w.shape
    bm = 1000

    def body(x_ref, w_ref, b_ref, o_ref):
        o_ref[...] = jnp.dot(x_ref[...], w_ref[...],
                             preferred_element_type=jnp.float32) + b_ref[...]

    return pl.pallas_call(
        body,
        grid=(m // bm,),
        in_specs=[
            pl.BlockSpec((bm, k), lambda i: (i, 0)),
            pl.BlockSpec((k, n), lambda i: (0, 0)),
            pl.BlockSpec((1, n), lambda i: (0, 0)),
        ],
        out_specs=pl.BlockSpec((bm, n), lambda i: (i, 0)),
        out_shape=jax.ShapeDtypeStruct((m, n), jnp.float32),
    )(x, w, b.reshape(1, n))


def _segment_softmax(logits, seg, num_segments):
    m = jax.ops.segment_max(logits, seg, num_segments=num_segments)
    m = jnp.where(jnp.isfinite(m), m, 0.0)
    e = jnp.exp(logits - m[seg])
    s = jax.ops.segment_sum(e, seg, num_segments=num_segments)
    return e / (s[seg] + 1e-16)


def _gat_conv(x, src, dst, W, att_src, att_dst, bias, heads, out_ch, n_nodes):
    xp = _matmul(x, W, bias * 0.0).reshape(n_nodes, heads, out_ch)
    a_src = (xp * att_src[None, :, :]).sum(-1)
    a_dst = (xp * att_dst[None, :, :]).sum(-1)
    alpha = a_src[src] + a_dst[dst]
    alpha = jax.nn.leaky_relu(alpha, negative_slope=0.2)
    alpha = _segment_softmax(alpha, dst, n_nodes)
    out = jax.ops.segment_sum(xp[src] * alpha[:, :, None], dst, num_segments=n_nodes)
    return out.reshape(n_nodes, heads * out_ch) + bias


def kernel(x, edge_index, W1, att_src1, att_dst1, b1, W2, att_src2, att_dst2, b2):
    loop = jnp.arange(N, dtype=edge_index.dtype)
    ei = jnp.concatenate([edge_index, jnp.stack([loop, loop])], axis=1)
    src, dst = ei[0], ei[1]
    h = _gat_conv(x, src, dst, W1, att_src1, att_dst1, b1, H1, C1, N)
    h = jax.nn.elu(h)
    out = _gat_conv(h, src, dst, W2, att_src2, att_dst2, b2, H2, C2, N)
    return out


# speedup vs baseline: 30.8125x; 30.8125x over previous
import dataclasses
import functools

import jax
import jax.numpy as jnp
from jax import lax
from jax.experimental import pallas as pl
from jax.experimental.pallas import tpu as pltpu
from jax.experimental.pallas import tpu_sc as plsc

N = 10000
E = 320000
F_IN = 128
H = 8
C1 = 64
C2 = 16

N_ACC = 10240
BM = 1024
CHUNK = 128
NSUB = 16
NCHUNK = 162
EPS = NSUB * NCHUNK * CHUNK
E_TOT = E + N

_MESH = dict(core_axis_name="c", subcore_axis_name="s")

_CP = pltpu.CompilerParams()
if "needs_layout_passes" in pltpu.CompilerParams.__dataclass_fields__:
    _CP = dataclasses.replace(_CP, needs_layout_passes=False)
if "use_tc_tiling_on_sc" in pltpu.CompilerParams.__dataclass_fields__:
    _CP = dataclasses.replace(_CP, use_tc_tiling_on_sc=False)

_f32 = jnp.float32
_i32 = jnp.int32


def _sds(shape, dtype=_f32):
    return jax.ShapeDtypeStruct(shape, dtype)


def _head_scalars(xp, att_src, att_dst, ch):
    cols_s, cols_d = [], []
    for h in range(H):
        xh = xp[:, h * ch:(h + 1) * ch]
        cols_s.append(jnp.sum(xh * att_src[h][None, :], axis=1, keepdims=True))
        cols_d.append(jnp.sum(xh * att_dst[h][None, :], axis=1, keepdims=True))
    asrc = jnp.concatenate(cols_s, axis=1)
    adst = jnp.concatenate(cols_d, axis=1)
    ad0 = jnp.concatenate([asrc[:, :4], adst[:, :4]], axis=1)
    ad1 = jnp.concatenate([asrc[:, 4:], adst[:, 4:]], axis=1)
    return ad0, ad1


def _tc_pre_body(x_ref, w1_ref, as_ref, ad_ref,
                 t0, t1, t2, t3, ad0_ref, ad1_ref):
    xp = jnp.dot(x_ref[...], w1_ref[...], preferred_element_type=_f32)
    for g, t in enumerate((t0, t1, t2, t3)):
        t[...] = xp[:, g * 128:(g + 1) * 128]
    ad0_ref[...], ad1_ref[...] = _head_scalars(xp, as_ref[...], ad_ref[...], C1)


def _tc_pre(x_pad, W1, att_src1, att_dst1):
    return pl.pallas_call(
        _tc_pre_body,
        grid=(N_ACC // BM,),
        in_specs=[
            pl.BlockSpec((BM, F_IN), lambda i: (i, 0)),
            pl.BlockSpec((F_IN, H * C1), lambda i: (0, 0)),
            pl.BlockSpec((H, C1), lambda i: (0, 0)),
            pl.BlockSpec((H, C1), lambda i: (0, 0)),
        ],
        out_specs=[pl.BlockSpec((BM, 128), lambda i: (i, 0))] * 4
        + [pl.BlockSpec((BM, 8), lambda i: (i, 0))] * 2,
        out_shape=[_sds((N_ACC, 128))] * 4 + [_sds((N_ACC, 8))] * 2,
    )(x_pad, W1, att_src1, att_dst1)


def _zero_rows(ref, nrow, ncol):
    z = jnp.zeros((16,), _f32)

    @pl.loop(0, nrow)
    def _(b):
        for j in range(ncol // 16):
            ref[b, pl.ds(j * 16, 16)] = z


def _compute_e(srow_v, drow_v, e_buf, e_srows, entries):
    iota = lax.iota(_i32, 16)
    for k in range(CHUNK // 16):
        idx16 = iota + (k * 16)
        for (jc, sl, dl, ol) in entries:
            av = plsc.load_gather(srow_v, [idx16, jnp.full((16,), sl, _i32)])
            bv = plsc.load_gather(drow_v, [idx16, jnp.full((16,), dl, _i32)])
            l = av + bv
            e16 = jnp.exp(jnp.maximum(l, l * 0.2))
            plsc.store_scatter(e_buf, [idx16, jnp.full((16,), jc, _i32)], e16)
            plsc.store_scatter(e_srows, [idx16, jnp.full((16,), ol, _i32)], e16)


def _sc_layer1(srcp, dstp, t0, t1, t2, t3, ad0, ad1):
    rps = N_ACC // NSUB

    @functools.partial(
        pl.kernel,
        out_type=[_sds((N_ACC, 128))] * 4 + [_sds((N_ACC, 16))] * 2,
        mesh=plsc.VectorSubcoreMesh(**_MESH),
        compiler_params=_CP,
        scratch_types=[
            pltpu.VMEM((CHUNK,), _i32),
            pltpu.VMEM((CHUNK,), _i32),
            pltpu.VMEM((CHUNK, 128), _f32),
            pltpu.VMEM((CHUNK, 8), _f32),
            pltpu.VMEM((CHUNK, 8), _f32),
            pltpu.VMEM((CHUNK, 2), _f32),
            pltpu.VMEM((CHUNK, 16), _f32),
            pltpu.VMEM_SHARED((N_ACC, 128), _f32),
            pltpu.VMEM_SHARED((N_ACC, 16), _f32),
            pltpu.SemaphoreType.DMA,
            pltpu.SemaphoreType.DMA,
            pltpu.SemaphoreType.DMA,
        ],
    )
    def k(src_hbm, dst_hbm, t0_hbm, t1_hbm, t2_hbm, t3_hbm, ad0_hbm, ad1_hbm,
          o0, o1, o2, o3, s0, s1,
          src_v, dst_v, rows_v, srow_v, drow_v, e_buf, e_srows,
          acc_sh, s_sh, sem, sem_s, sem_d):
        c = lax.axis_index("c")
        s = lax.axis_index("s")
        ebase = s * (NCHUNK * CHUNK)
        rbase = s * rps
        outs = (o0, o1, o2, o3)
        tabs = (t0_hbm, t1_hbm, t2_hbm, t3_hbm)

        for gpass in range(2):
            _zero_rows(rows_v, CHUNK, 128)
            _zero_rows(e_srows, CHUNK, 16)
            for kk in range(rps // CHUNK):
                sl = pl.ds(rbase + kk * CHUNK, CHUNK)
                pltpu.sync_copy(rows_v, acc_sh.at[sl])
                if gpass == 0:
                    pltpu.sync_copy(e_srows, s_sh.at[sl])
            plsc.subcore_barrier()

            @pl.loop(0, NCHUNK)
            def _(i):
                base = ebase + i * CHUNK
                pltpu.sync_copy(src_hbm.at[pl.ds(base, CHUNK)], src_v)
                pltpu.sync_copy(dst_hbm.at[pl.ds(base, CHUNK)], dst_v)

                @pl.when(c == 0)
                def _():
                    pltpu.async_copy(tabs[gpass].at[src_v], rows_v, sem)
                    pltpu.async_copy(ad0_hbm.at[src_v], srow_v, sem_s)
                    pltpu.async_copy(ad0_hbm.at[dst_v], drow_v, sem_d)

                @pl.when(c == 1)
                def _():
                    pltpu.async_copy(tabs[2 + gpass].at[src_v], rows_v, sem)
                    pltpu.async_copy(ad1_hbm.at[src_v], srow_v, sem_s)
                    pltpu.async_copy(ad1_hbm.at[dst_v], drow_v, sem_d)

                pltpu.make_async_copy(ad0_hbm.at[src_v], srow_v, sem_s).wait()
                pltpu.make_async_copy(ad0_hbm.at[dst_v], drow_v, sem_d).wait()

                _compute_e(srow_v, drow_v, e_buf, e_srows,
                           [(0, 2 * gpass, 4 + 2 * gpass, 2 * gpass),
                            (1, 2 * gpass + 1, 5 + 2 * gpass, 2 * gpass + 1)])
                pltpu.sync_copy(e_srows, s_sh.at[dst_v], add=True)
                pltpu.make_async_copy(tabs[gpass].at[src_v], rows_v, sem).wait()

                zero16 = jnp.zeros((16,), _i32)
                one16 = jnp.full((16,), 1, _i32)

                @pl.loop(0, CHUNK)
                def _(b):
                    bsp = jnp.full((16,), b, _i32)
                    e0 = plsc.load_gather(e_buf, [bsp, zero16])
                    e1 = plsc.load_gather(e_buf, [bsp, one16])
                    for j in range(8):
                        ev = e0 if j < 4 else e1
                        sl = pl.ds(j * 16, 16)
                        rows_v[b, sl] = rows_v[b, sl] * ev

                pltpu.sync_copy(rows_v, acc_sh.at[dst_v], add=True)

            plsc.subcore_barrier()
            for kk in range(rps // CHUNK):
                sl = pl.ds(rbase + kk * CHUNK, CHUNK)

                @pl.when(c == 0)
                def _():
                    pltpu.sync_copy(acc_sh.at[sl], outs[gpass].at[sl])

                @pl.when(c == 1)
                def _():
                    pltpu.sync_copy(acc_sh.at[sl], outs[2 + gpass].at[sl])

            plsc.subcore_barrier()

        for kk in range(rps // CHUNK):
            sl = pl.ds(rbase + kk * CHUNK, CHUNK)

            @pl.when(c == 0)
            def _():
                pltpu.sync_copy(s_sh.at[sl], s0.at[sl])

            @pl.when(c == 1)
            def _():
                pltpu.sync_copy(s_sh.at[sl], s1.at[sl])

    return k(srcp, dstp, t0, t1, t2, t3, ad0, ad1)


def _sc_layer2(srcp, dstp, tA, tB, ad0, ad1):
    rps = N_ACC // NSUB

    @functools.partial(
        pl.kernel,
        out_type=[_sds((N_ACC, 64))] * 2 + [_sds((N_ACC, 16))] * 2,
        mesh=plsc.VectorSubcoreMesh(**_MESH),
        compiler_params=_CP,
        scratch_types=[
            pltpu.VMEM((CHUNK,), _i32),
            pltpu.VMEM((CHUNK,), _i32),
            pltpu.VMEM((CHUNK, 64), _f32),
            pltpu.VMEM((CHUNK, 8), _f32),
            pltpu.VMEM((CHUNK, 8), _f32),
            pltpu.VMEM((CHUNK, 4), _f32),
            pltpu.VMEM((CHUNK, 16), _f32),
            pltpu.VMEM_SHARED((N_ACC, 64), _f32),
            pltpu.VMEM_SHARED((N_ACC, 16), _f32),
            pltpu.SemaphoreType.DMA,
            pltpu.SemaphoreType.DMA,
            pltpu.SemaphoreType.DMA,
        ],
    )
    def k(src_hbm, dst_hbm, ta_hbm, tb_hbm, ad0_hbm, ad1_hbm,
          oA, oB, s0, s1,
          src_v, dst_v, rows_v, srow_v, drow_v, e_buf, e_srows,
          acc_sh, s_sh, sem, sem_s, sem_d):
        c = lax.axis_index("c")
        s = lax.axis_index("s")
        ebase = s * (NCHUNK * CHUNK)
        rbase = s * rps

        _zero_rows(rows_v, CHUNK, 64)
        _zero_rows(e_srows, CHUNK, 16)
        for kk in range(rps // CHUNK):
            sl = pl.ds(rbase + kk * CHUNK, CHUNK)
            pltpu.sync_copy(rows_v, acc_sh.at[sl])
            pltpu.sync_copy(e_srows, s_sh.at[sl])
        plsc.subcore_barrier()

        @pl.loop(0, NCHUNK)
        def _(i):
            base = ebase + i * CHUNK
            pltpu.sync_copy(src_hbm.at[pl.ds(base, CHUNK)], src_v)
            pltpu.sync_copy(dst_hbm.at[pl.ds(base, CHUNK)], dst_v)

            @pl.when(c == 0)
            def _():
                pltpu.async_copy(ta_hbm.at[src_v], rows_v, sem)
                pltpu.async_copy(ad0_hbm.at[src_v], srow_v, sem_s)
                pltpu.async_copy(ad0_hbm.at[dst_v], drow_v, sem_d)

            @pl.when(c == 1)
            def _():
                pltpu.async_copy(tb_hbm.at[src_v], rows_v, sem)
                pltpu.async_copy(ad1_hbm.at[src_v], srow_v, sem_s)
                pltpu.async_copy(ad1_hbm.at[dst_v], drow_v, sem_d)

            pltpu.make_async_copy(ad0_hbm.at[src_v], srow_v, sem_s).wait()
            pltpu.make_async_copy(ad0_hbm.at[dst_v], drow_v, sem_d).wait()

            _compute_e(srow_v, drow_v, e_buf, e_srows,
                       [(j, j, 4 + j, j) for j in range(4)])
            pltpu.sync_copy(e_srows, s_sh.at[dst_v], add=True)
            pltpu.make_async_copy(ta_hbm.at[src_v], rows_v, sem).wait()

            @pl.loop(0, CHUNK)
            def _(b):
                bsp = jnp.full((16,), b, _i32)
                for j in range(4):
                    ev = plsc.load_gather(e_buf, [bsp, jnp.full((16,), j, _i32)])
                    sl = pl.ds(j * 16, 16)
                    rows_v[b, sl] = rows_v[b, sl] * ev

            pltpu.sync_copy(rows_v, acc_sh.at[dst_v], add=True)

        plsc.subcore_barrier()
        for kk in range(rps // CHUNK):
            sl = pl.ds(rbase + kk * CHUNK, CHUNK)

            @pl.when(c == 0)
            def _():
                pltpu.sync_copy(acc_sh.at[sl], oA.at[sl])
                pltpu.sync_copy(s_sh.at[sl], s0.at[sl])

            @pl.when(c == 1)
            def _():
                pltpu.sync_copy(acc_sh.at[sl], oB.at[sl])
                pltpu.sync_copy(s_sh.at[sl], s1.at[sl])

    return k(srcp, dstp, tA, tB, ad0, ad1)


def _tc_mid_body(a0, a1, a2, a3, s0_ref, s1_ref, b1_ref, w2_ref,
                 as_ref, ad_ref, tA, tB, ad0_ref, ad1_ref):
    parts = []
    for g in range(4):
        a = (a0, a1, a2, a3)[g][...]
        s_ref = s0_ref if g < 2 else s1_ref
        for d in range(2):
            h = 2 * g + d
            lane = h - 4 * (h // 4)
            den = s_ref[:, lane:lane + 1] + 1e-16
            parts.append(a[:, d * 64:(d + 1) * 64] / den)
    h1 = jnp.concatenate(parts, axis=1) + b1_ref[...]
    h1 = jnp.where(h1 > 0, h1, jnp.exp(h1) - 1.0)
    xp2 = jnp.dot(h1, w2_ref[...], preferred_element_type=_f32)
    tA[...] = xp2[:, :64]
    tB[...] = xp2[:, 64:]
    ad0_ref[...], ad1_ref[...] = _head_scalars(xp2, as_ref[...], ad_ref[...], C2)


def _tc_mid(accs, s0, s1, b1, W2, att_src2, att_dst2):
    return pl.pallas_call(
        _tc_mid_body,
        grid=(N_ACC // BM,),
        in_specs=[pl.BlockSpec((BM, 128), lambda i: (i, 0))] * 4
        + [pl.BlockSpec((BM, 16), lambda i: (i, 0))] * 2
        + [
            pl.BlockSpec((1, H * C1), lambda i: (0, 0)),
            pl.BlockSpec((H * C1, H * C2), lambda i: (0, 0)),
            pl.BlockSpec((H, C2), lambda i: (0, 0)),
            pl.BlockSpec((H, C2), lambda i: (0, 0)),
        ],
        out_specs=[pl.BlockSpec((BM, 64), lambda i: (i, 0))] * 2
        + [pl.BlockSpec((BM, 8), lambda i: (i, 0))] * 2,
        out_shape=[_sds((N_ACC, 64))] * 2 + [_sds((N_ACC, 8))] * 2,
    )(*accs, s0, s1, b1.reshape(1, -1), W2, att_src2, att_dst2)


def _tc_post_body(aA, aB, s0_ref, s1_ref, b2_ref, o_ref):
    parts = []
    for h in range(H):
        acc = (aA if h < 4 else aB)[...]
        s_ref = s0_ref if h < 4 else s1_ref
        j = h % 4
        den = s_ref[:, j:j + 1] + 1e-16
        parts.append(acc[:, j * 16:(j + 1) * 16] / den)
    o_ref[...] = jnp.concatenate(parts, axis=1) + b2_ref[...]


def _tc_post(aA, aB, s0, s1, b2):
    bm = 1000
    return pl.pallas_call(
        _tc_post_body,
        grid=(N // bm,),
        in_specs=[pl.BlockSpec((bm, 64), lambda i: (i, 0))] * 2
        + [pl.BlockSpec((bm, 16), lambda i: (i, 0))] * 2
        + [pl.BlockSpec((1, H * C2), lambda i: (0, 0))],
        out_specs=pl.BlockSpec((bm, H * C2), lambda i: (i, 0)),
        out_shape=_sds((N, H * C2)),
    )(aA, aB, s0, s1, b2.reshape(1, -1))


def kernel(x, edge_index, W1, att_src1, att_dst1, b1, W2, att_src2, att_dst2, b2):
    loop = jnp.arange(N, dtype=jnp.int32)
    pad = jnp.full((EPS - E_TOT,), N, jnp.int32)
    srcp = jnp.concatenate([edge_index[0].astype(jnp.int32), loop, pad])
    dstp = jnp.concatenate([edge_index[1].astype(jnp.int32), loop, pad])
    x_pad = jnp.pad(x, ((0, N_ACC - N), (0, 0)))

    t0, t1, t2, t3, ad1_0, ad1_1 = _tc_pre(x_pad, W1, att_src1, att_dst1)
    a0, a1, a2, a3, s0, s1 = _sc_layer1(srcp, dstp, t0, t1, t2, t3, ad1_0, ad1_1)
    tA, tB, ad2_0, ad2_1 = _tc_mid((a0, a1, a2, a3), s0, s1, b1, W2,
                                   att_src2, att_dst2)
    aA, aB, s20, s21 = _sc_layer2(srcp, dstp, tA, tB, ad2_0, ad2_1)
    return _tc_post(aA, aB, s20, s21, b2)


# speedup vs baseline: 37.4560x; 1.2156x over previous
import dataclasses
import functools

import jax
import jax.numpy as jnp
from jax import lax
from jax.experimental import pallas as pl
from jax.experimental.pallas import tpu as pltpu
from jax.experimental.pallas import tpu_sc as plsc

N = 10000
E = 320000
F_IN = 128
H = 8
C1 = 64
C2 = 16

N_ACC = 10240
BM = 1024
CHUNK = 128
NSUB = 16
NCHUNK = 162
EPS = NSUB * NCHUNK * CHUNK
E_TOT = E + N

_MESH = dict(core_axis_name="c", subcore_axis_name="s")

_CP = pltpu.CompilerParams()
if "needs_layout_passes" in pltpu.CompilerParams.__dataclass_fields__:
    _CP = dataclasses.replace(_CP, needs_layout_passes=False)
if "use_tc_tiling_on_sc" in pltpu.CompilerParams.__dataclass_fields__:
    _CP = dataclasses.replace(_CP, use_tc_tiling_on_sc=False)

_f32 = jnp.float32
_i32 = jnp.int32


def _sds(shape, dtype=_f32):
    return jax.ShapeDtypeStruct(shape, dtype)


def _head_scalars(xp, att_src, att_dst, ch):
    cols_s, cols_d = [], []
    for h in range(H):
        xh = xp[:, h * ch:(h + 1) * ch]
        cols_s.append(jnp.sum(xh * att_src[h][None, :], axis=1, keepdims=True))
        cols_d.append(jnp.sum(xh * att_dst[h][None, :], axis=1, keepdims=True))
    asrc = jnp.concatenate(cols_s, axis=1)
    adst = jnp.concatenate(cols_d, axis=1)
    ad0 = jnp.concatenate([asrc[:, :4], adst[:, :4]], axis=1)
    ad1 = jnp.concatenate([asrc[:, 4:], adst[:, 4:]], axis=1)
    return ad0, ad1


def _tc_pre_body(x_ref, w1_ref, as_ref, ad_ref,
                 t0, t1, t2, t3, ad0_ref, ad1_ref):
    xp = jnp.dot(x_ref[...], w1_ref[...], preferred_element_type=_f32)
    for g, t in enumerate((t0, t1, t2, t3)):
        t[...] = xp[:, g * 128:(g + 1) * 128]
    ad0_ref[...], ad1_ref[...] = _head_scalars(xp, as_ref[...], ad_ref[...], C1)


def _tc_pre(x_pad, W1, att_src1, att_dst1):
    return pl.pallas_call(
        _tc_pre_body,
        grid=(N_ACC // BM,),
        in_specs=[
            pl.BlockSpec((BM, F_IN), lambda i: (i, 0)),
            pl.BlockSpec((F_IN, H * C1), lambda i: (0, 0)),
            pl.BlockSpec((H, C1), lambda i: (0, 0)),
            pl.BlockSpec((H, C1), lambda i: (0, 0)),
        ],
        out_specs=[pl.BlockSpec((BM, 128), lambda i: (i, 0))] * 4
        + [pl.BlockSpec((BM, 8), lambda i: (i, 0))] * 2,
        out_shape=[_sds((N_ACC, 128))] * 4 + [_sds((N_ACC, 8))] * 2,
    )(x_pad, W1, att_src1, att_dst1)


def _zero_rows(ref, nrow, ncol):
    z = jnp.zeros((16,), _f32)

    @pl.loop(0, nrow)
    def _(b):
        for j in range(ncol // 16):
            ref[b, pl.ds(j * 16, 16)] = z


def _compute_e(srow_v, drow_v, e_buf, e_srows, entries):
    iota = lax.iota(_i32, 16)
    for k in range(CHUNK // 16):
        idx16 = iota + (k * 16)
        for (jc, sl, dl, ol) in entries:
            av = plsc.load_gather(srow_v, [idx16, jnp.full((16,), sl, _i32)])
            bv = plsc.load_gather(drow_v, [idx16, jnp.full((16,), dl, _i32)])
            l = av + bv
            e16 = jnp.exp(jnp.maximum(l, l * 0.2))
            plsc.store_scatter(e_buf, [idx16, jnp.full((16,), jc, _i32)], e16)
            plsc.store_scatter(e_srows, [idx16, jnp.full((16,), ol, _i32)], e16)


def _sc_layer1(srcp, dstp, t0, t1, t2, t3, ad0, ad1):
    rps = N_ACC // NSUB

    @functools.partial(
        pl.kernel,
        out_type=[_sds((N_ACC, 128))] * 4 + [_sds((N_ACC, 16))] * 2,
        mesh=plsc.VectorSubcoreMesh(**_MESH),
        compiler_params=_CP,
        scratch_types=[
            pltpu.VMEM((CHUNK,), _i32),
            pltpu.VMEM((CHUNK,), _i32),
            pltpu.VMEM((CHUNK, 128), _f32),
            pltpu.VMEM((CHUNK, 8), _f32),
            pltpu.VMEM((CHUNK, 8), _f32),
            pltpu.VMEM((CHUNK, 2), _f32),
            pltpu.VMEM((CHUNK, 16), _f32),
            pltpu.VMEM_SHARED((N_ACC, 128), _f32),
            pltpu.VMEM_SHARED((N_ACC, 16), _f32),
            pltpu.SemaphoreType.DMA,
            pltpu.SemaphoreType.DMA,
            pltpu.SemaphoreType.DMA,
        ],
    )
    def k(src_hbm, dst_hbm, t0_hbm, t1_hbm, t2_hbm, t3_hbm, ad0_hbm, ad1_hbm,
          o0, o1, o2, o3, s0, s1,
          src_v, dst_v, rows_v, srow_v, drow_v, e_buf, e_srows,
          acc_sh, s_sh, sem, sem_s, sem_d):
        c = lax.axis_index("c")
        s = lax.axis_index("s")
        ebase = s * (NCHUNK * CHUNK)
        rbase = s * rps
        outs = (o0, o1, o2, o3)
        tabs = (t0_hbm, t1_hbm, t2_hbm, t3_hbm)

        for gpass in range(2):
            _zero_rows(rows_v, CHUNK, 128)
            _zero_rows(e_srows, CHUNK, 16)
            for kk in range(rps // CHUNK):
                sl = pl.ds(rbase + kk * CHUNK, CHUNK)
                pltpu.sync_copy(rows_v, acc_sh.at[sl])
                if gpass == 0:
                    pltpu.sync_copy(e_srows, s_sh.at[sl])
            plsc.subcore_barrier()

            @pl.loop(0, NCHUNK)
            def _(i):
                base = ebase + i * CHUNK
                pltpu.sync_copy(src_hbm.at[pl.ds(base, CHUNK)], src_v)
                pltpu.sync_copy(dst_hbm.at[pl.ds(base, CHUNK)], dst_v)

                @pl.when(c == 0)
                def _():
                    pltpu.async_copy(tabs[gpass].at[src_v], rows_v, sem)
                    pltpu.async_copy(ad0_hbm.at[src_v], srow_v, sem_s)
                    pltpu.async_copy(ad0_hbm.at[dst_v], drow_v, sem_d)

                @pl.when(c == 1)
                def _():
                    pltpu.async_copy(tabs[2 + gpass].at[src_v], rows_v, sem)
                    pltpu.async_copy(ad1_hbm.at[src_v], srow_v, sem_s)
                    pltpu.async_copy(ad1_hbm.at[dst_v], drow_v, sem_d)

                pltpu.make_async_copy(ad0_hbm.at[src_v], srow_v, sem_s).wait()
                pltpu.make_async_copy(ad0_hbm.at[dst_v], drow_v, sem_d).wait()

                _compute_e(srow_v, drow_v, e_buf, e_srows,
                           [(0, 2 * gpass, 4 + 2 * gpass, 2 * gpass),
                            (1, 2 * gpass + 1, 5 + 2 * gpass, 2 * gpass + 1)])
                pltpu.sync_copy(e_srows, s_sh.at[dst_v], add=True)
                pltpu.make_async_copy(tabs[gpass].at[src_v], rows_v, sem).wait()

                zero16 = jnp.zeros((16,), _i32)
                one16 = jnp.full((16,), 1, _i32)

                @plsc.parallel_loop(0, CHUNK, unroll=4)
                def _(b):
                    bsp = jnp.full((16,), b, _i32)
                    e0 = plsc.load_gather(e_buf, [bsp, zero16])
                    e1 = plsc.load_gather(e_buf, [bsp, one16])
                    for j in range(8):
                        ev = e0 if j < 4 else e1
                        sl = pl.ds(j * 16, 16)
                        rows_v[b, sl] = rows_v[b, sl] * ev

                pltpu.sync_copy(rows_v, acc_sh.at[dst_v], add=True)

            plsc.subcore_barrier()
            for kk in range(rps // CHUNK):
                sl = pl.ds(rbase + kk * CHUNK, CHUNK)

                @pl.when(c == 0)
                def _():
                    pltpu.sync_copy(acc_sh.at[sl], outs[gpass].at[sl])

                @pl.when(c == 1)
                def _():
                    pltpu.sync_copy(acc_sh.at[sl], outs[2 + gpass].at[sl])

            plsc.subcore_barrier()

        for kk in range(rps // CHUNK):
            sl = pl.ds(rbase + kk * CHUNK, CHUNK)

            @pl.when(c == 0)
            def _():
                pltpu.sync_copy(s_sh.at[sl], s0.at[sl])

            @pl.when(c == 1)
            def _():
                pltpu.sync_copy(s_sh.at[sl], s1.at[sl])

    return k(srcp, dstp, t0, t1, t2, t3, ad0, ad1)


def _sc_layer2(srcp, dstp, tA, tB, ad0, ad1):
    rps = N_ACC // NSUB

    @functools.partial(
        pl.kernel,
        out_type=[_sds((N_ACC, 64))] * 2 + [_sds((N_ACC, 16))] * 2,
        mesh=plsc.VectorSubcoreMesh(**_MESH),
        compiler_params=_CP,
        scratch_types=[
            pltpu.VMEM((CHUNK,), _i32),
            pltpu.VMEM((CHUNK,), _i32),
            pltpu.VMEM((CHUNK, 64), _f32),
            pltpu.VMEM((CHUNK, 8), _f32),
            pltpu.VMEM((CHUNK, 8), _f32),
            pltpu.VMEM((CHUNK, 4), _f32),
            pltpu.VMEM((CHUNK, 16), _f32),
            pltpu.VMEM_SHARED((N_ACC, 64), _f32),
            pltpu.VMEM_SHARED((N_ACC, 16), _f32),
            pltpu.SemaphoreType.DMA,
            pltpu.SemaphoreType.DMA,
            pltpu.SemaphoreType.DMA,
        ],
    )
    def k(src_hbm, dst_hbm, ta_hbm, tb_hbm, ad0_hbm, ad1_hbm,
          oA, oB, s0, s1,
          src_v, dst_v, rows_v, srow_v, drow_v, e_buf, e_srows,
          acc_sh, s_sh, sem, sem_s, sem_d):
        c = lax.axis_index("c")
        s = lax.axis_index("s")
        ebase = s * (NCHUNK * CHUNK)
        rbase = s * rps

        _zero_rows(rows_v, CHUNK, 64)
        _zero_rows(e_srows, CHUNK, 16)
        for kk in range(rps // CHUNK):
            sl = pl.ds(rbase + kk * CHUNK, CHUNK)
            pltpu.sync_copy(rows_v, acc_sh.at[sl])
            pltpu.sync_copy(e_srows, s_sh.at[sl])
        plsc.subcore_barrier()

        @pl.loop(0, NCHUNK)
        def _(i):
            base = ebase + i * CHUNK
            pltpu.sync_copy(src_hbm.at[pl.ds(base, CHUNK)], src_v)
            pltpu.sync_copy(dst_hbm.at[pl.ds(base, CHUNK)], dst_v)

            @pl.when(c == 0)
            def _():
                pltpu.async_copy(ta_hbm.at[src_v], rows_v, sem)
                pltpu.async_copy(ad0_hbm.at[src_v], srow_v, sem_s)
                pltpu.async_copy(ad0_hbm.at[dst_v], drow_v, sem_d)

            @pl.when(c == 1)
            def _():
                pltpu.async_copy(tb_hbm.at[src_v], rows_v, sem)
                pltpu.async_copy(ad1_hbm.at[src_v], srow_v, sem_s)
                pltpu.async_copy(ad1_hbm.at[dst_v], drow_v, sem_d)

            pltpu.make_async_copy(ad0_hbm.at[src_v], srow_v, sem_s).wait()
            pltpu.make_async_copy(ad0_hbm.at[dst_v], drow_v, sem_d).wait()

            _compute_e(srow_v, drow_v, e_buf, e_srows,
                       [(j, j, 4 + j, j) for j in range(4)])
            pltpu.sync_copy(e_srows, s_sh.at[dst_v], add=True)
            pltpu.make_async_copy(ta_hbm.at[src_v], rows_v, sem).wait()

            @plsc.parallel_loop(0, CHUNK, unroll=4)
            def _(b):
                bsp = jnp.full((16,), b, _i32)
                for j in range(4):
                    ev = plsc.load_gather(e_buf, [bsp, jnp.full((16,), j, _i32)])
                    sl = pl.ds(j * 16, 16)
                    rows_v[b, sl] = rows_v[b, sl] * ev

            pltpu.sync_copy(rows_v, acc_sh.at[dst_v], add=True)

        plsc.subcore_barrier()
        for kk in range(rps // CHUNK):
            sl = pl.ds(rbase + kk * CHUNK, CHUNK)

            @pl.when(c == 0)
            def _():
                pltpu.sync_copy(acc_sh.at[sl], oA.at[sl])
                pltpu.sync_copy(s_sh.at[sl], s0.at[sl])

            @pl.when(c == 1)
            def _():
                pltpu.sync_copy(acc_sh.at[sl], oB.at[sl])
                pltpu.sync_copy(s_sh.at[sl], s1.at[sl])

    return k(srcp, dstp, tA, tB, ad0, ad1)


def _tc_mid_body(a0, a1, a2, a3, s0_ref, s1_ref, b1_ref, w2_ref,
                 as_ref, ad_ref, tA, tB, ad0_ref, ad1_ref):
    parts = []
    for g in range(4):
        a = (a0, a1, a2, a3)[g][...]
        s_ref = s0_ref if g < 2 else s1_ref
        for d in range(2):
            h = 2 * g + d
            lane = h - 4 * (h // 4)
            den = s_ref[:, lane:lane + 1] + 1e-16
            parts.append(a[:, d * 64:(d + 1) * 64] / den)
    h1 = jnp.concatenate(parts, axis=1) + b1_ref[...]
    h1 = jnp.where(h1 > 0, h1, jnp.exp(h1) - 1.0)
    xp2 = jnp.dot(h1, w2_ref[...], preferred_element_type=_f32)
    tA[...] = xp2[:, :64]
    tB[...] = xp2[:, 64:]
    ad0_ref[...], ad1_ref[...] = _head_scalars(xp2, as_ref[...], ad_ref[...], C2)


def _tc_mid(accs, s0, s1, b1, W2, att_src2, att_dst2):
    return pl.pallas_call(
        _tc_mid_body,
        grid=(N_ACC // BM,),
        in_specs=[pl.BlockSpec((BM, 128), lambda i: (i, 0))] * 4
        + [pl.BlockSpec((BM, 16), lambda i: (i, 0))] * 2
        + [
            pl.BlockSpec((1, H * C1), lambda i: (0, 0)),
            pl.BlockSpec((H * C1, H * C2), lambda i: (0, 0)),
            pl.BlockSpec((H, C2), lambda i: (0, 0)),
            pl.BlockSpec((H, C2), lambda i: (0, 0)),
        ],
        out_specs=[pl.BlockSpec((BM, 64), lambda i: (i, 0))] * 2
        + [pl.BlockSpec((BM, 8), lambda i: (i, 0))] * 2,
        out_shape=[_sds((N_ACC, 64))] * 2 + [_sds((N_ACC, 8))] * 2,
    )(*accs, s0, s1, b1.reshape(1, -1), W2, att_src2, att_dst2)


def _tc_post_body(aA, aB, s0_ref, s1_ref, b2_ref, o_ref):
    parts = []
    for h in range(H):
        acc = (aA if h < 4 else aB)[...]
        s_ref = s0_ref if h < 4 else s1_ref
        j = h % 4
        den = s_ref[:, j:j + 1] + 1e-16
        parts.append(acc[:, j * 16:(j + 1) * 16] / den)
    o_ref[...] = jnp.concatenate(parts, axis=1) + b2_ref[...]


def _tc_post(aA, aB, s0, s1, b2):
    bm = 1000
    return pl.pallas_call(
        _tc_post_body,
        grid=(N // bm,),
        in_specs=[pl.BlockSpec((bm, 64), lambda i: (i, 0))] * 2
        + [pl.BlockSpec((bm, 16), lambda i: (i, 0))] * 2
        + [pl.BlockSpec((1, H * C2), lambda i: (0, 0))],
        out_specs=pl.BlockSpec((bm, H * C2), lambda i: (i, 0)),
        out_shape=_sds((N, H * C2)),
    )(aA, aB, s0, s1, b2.reshape(1, -1))


def kernel(x, edge_index, W1, att_src1, att_dst1, b1, W2, att_src2, att_dst2, b2):
    loop = jnp.arange(N, dtype=jnp.int32)
    pad = jnp.full((EPS - E_TOT,), N, jnp.int32)
    srcp = jnp.concatenate([edge_index[0].astype(jnp.int32), loop, pad])
    dstp = jnp.concatenate([edge_index[1].astype(jnp.int32), loop, pad])
    x_pad = jnp.pad(x, ((0, N_ACC - N), (0, 0)))

    t0, t1, t2, t3, ad1_0, ad1_1 = _tc_pre(x_pad, W1, att_src1, att_dst1)
    a0, a1, a2, a3, s0, s1 = _sc_layer1(srcp, dstp, t0, t1, t2, t3, ad1_0, ad1_1)
    tA, tB, ad2_0, ad2_1 = _tc_mid((a0, a1, a2, a3), s0, s1, b1, W2,
                                   att_src2, att_dst2)
    aA, aB, s20, s21 = _sc_layer2(srcp, dstp, tA, tB, ad2_0, ad2_1)
    return _tc_post(aA, aB, s20, s21, b2)


# speedup vs baseline: 38.3837x; 1.0248x over previous
import dataclasses
import functools

import jax
import jax.numpy as jnp
from jax import lax
from jax.experimental import pallas as pl
from jax.experimental.pallas import tpu as pltpu
from jax.experimental.pallas import tpu_sc as plsc

N = 10000
E = 320000
F_IN = 128
H = 8
C1 = 64
C2 = 16

N_ACC = 10240
BM = 1024
CHUNK = 128
NSUB = 16
NCHUNK = 162
EPS = NSUB * NCHUNK * CHUNK
E_TOT = E + N

_MESH = dict(core_axis_name="c", subcore_axis_name="s")

_CP = pltpu.CompilerParams()
if "needs_layout_passes" in pltpu.CompilerParams.__dataclass_fields__:
    _CP = dataclasses.replace(_CP, needs_layout_passes=False)
if "use_tc_tiling_on_sc" in pltpu.CompilerParams.__dataclass_fields__:
    _CP = dataclasses.replace(_CP, use_tc_tiling_on_sc=False)

_f32 = jnp.float32
_i32 = jnp.int32


def _sds(shape, dtype=_f32):
    return jax.ShapeDtypeStruct(shape, dtype)


def _head_scalars(xp, att_src, att_dst, ch):
    cols_s, cols_d = [], []
    for h in range(H):
        xh = xp[:, h * ch:(h + 1) * ch]
        cols_s.append(jnp.sum(xh * att_src[h][None, :], axis=1, keepdims=True))
        cols_d.append(jnp.sum(xh * att_dst[h][None, :], axis=1, keepdims=True))
    asrc = jnp.concatenate(cols_s, axis=1)
    adst = jnp.concatenate(cols_d, axis=1)
    ad0 = jnp.concatenate([asrc[:, :4], adst[:, :4]], axis=1)
    ad1 = jnp.concatenate([asrc[:, 4:], adst[:, 4:]], axis=1)
    return ad0, ad1


def _tc_pre_body(x_ref, w1_ref, as_ref, ad_ref,
                 t0, t1, t2, t3, ad0_ref, ad1_ref):
    xp = jnp.dot(x_ref[...], w1_ref[...], preferred_element_type=_f32)
    for g, t in enumerate((t0, t1, t2, t3)):
        t[...] = xp[:, g * 128:(g + 1) * 128]
    ad0_ref[...], ad1_ref[...] = _head_scalars(xp, as_ref[...], ad_ref[...], C1)


def _tc_pre(x_pad, W1, att_src1, att_dst1):
    return pl.pallas_call(
        _tc_pre_body,
        grid=(N_ACC // BM,),
        in_specs=[
            pl.BlockSpec((BM, F_IN), lambda i: (i, 0)),
            pl.BlockSpec((F_IN, H * C1), lambda i: (0, 0)),
            pl.BlockSpec((H, C1), lambda i: (0, 0)),
            pl.BlockSpec((H, C1), lambda i: (0, 0)),
        ],
        out_specs=[pl.BlockSpec((BM, 128), lambda i: (i, 0))] * 4
        + [pl.BlockSpec((BM, 8), lambda i: (i, 0))] * 2,
        out_shape=[_sds((N_ACC, 128))] * 4 + [_sds((N_ACC, 8))] * 2,
    )(x_pad, W1, att_src1, att_dst1)


def _zero_rows(ref, nrow, ncol):
    z = jnp.zeros((16,), _f32)

    @pl.loop(0, nrow)
    def _(b):
        for j in range(ncol // 16):
            ref[b, pl.ds(j * 16, 16)] = z


def _compute_e(srow_v, drow_v, e_buf, e_srows, entries):
    iota = lax.iota(_i32, 16)

    @plsc.parallel_loop(0, CHUNK // 16, unroll=2)
    def _(k):
        idx16 = iota + k * 16
        for (jc, sl, dl, ol) in entries:
            av = plsc.load_gather(srow_v, [idx16, jnp.full((16,), sl, _i32)])
            bv = plsc.load_gather(drow_v, [idx16, jnp.full((16,), dl, _i32)])
            l = av + bv
            e16 = jnp.exp(jnp.maximum(l, l * 0.2))
            plsc.store_scatter(e_buf, [idx16, jnp.full((16,), jc, _i32)], e16)
            plsc.store_scatter(e_srows, [idx16, jnp.full((16,), ol, _i32)], e16)


def _sc_layer1(srcp, dstp, t0, t1, t2, t3, ad0, ad1):
    rps = N_ACC // NSUB

    @functools.partial(
        pl.kernel,
        out_type=[_sds((N_ACC, 128))] * 4 + [_sds((N_ACC, 16))] * 2,
        mesh=plsc.VectorSubcoreMesh(**_MESH),
        compiler_params=_CP,
        scratch_types=[
            pltpu.VMEM((CHUNK,), _i32),
            pltpu.VMEM((CHUNK,), _i32),
            pltpu.VMEM((CHUNK, 128), _f32),
            pltpu.VMEM((CHUNK, 8), _f32),
            pltpu.VMEM((CHUNK, 8), _f32),
            pltpu.VMEM((CHUNK, 2), _f32),
            pltpu.VMEM((CHUNK, 16), _f32),
            pltpu.VMEM_SHARED((N_ACC, 128), _f32),
            pltpu.VMEM_SHARED((N_ACC, 16), _f32),
            pltpu.SemaphoreType.DMA,
            pltpu.SemaphoreType.DMA,
            pltpu.SemaphoreType.DMA,
        ],
    )
    def k(src_hbm, dst_hbm, t0_hbm, t1_hbm, t2_hbm, t3_hbm, ad0_hbm, ad1_hbm,
          o0, o1, o2, o3, s0, s1,
          src_v, dst_v, rows_v, srow_v, drow_v, e_buf, e_srows,
          acc_sh, s_sh, sem, sem_s, sem_d):
        c = lax.axis_index("c")
        s = lax.axis_index("s")
        ebase = s * (NCHUNK * CHUNK)
        rbase = s * rps
        outs = (o0, o1, o2, o3)
        tabs = (t0_hbm, t1_hbm, t2_hbm, t3_hbm)

        for gpass in range(2):
            _zero_rows(rows_v, CHUNK, 128)
            _zero_rows(e_srows, CHUNK, 16)
            for kk in range(rps // CHUNK):
                sl = pl.ds(rbase + kk * CHUNK, CHUNK)
                pltpu.sync_copy(rows_v, acc_sh.at[sl])
                if gpass == 0:
                    pltpu.sync_copy(e_srows, s_sh.at[sl])
            plsc.subcore_barrier()

            @pl.loop(0, NCHUNK)
            def _(i):
                base = ebase + i * CHUNK
                pltpu.sync_copy(src_hbm.at[pl.ds(base, CHUNK)], src_v)
                pltpu.sync_copy(dst_hbm.at[pl.ds(base, CHUNK)], dst_v)

                @pl.when(c == 0)
                def _():
                    pltpu.async_copy(tabs[gpass].at[src_v], rows_v, sem)
                    pltpu.async_copy(ad0_hbm.at[src_v], srow_v, sem_s)
                    pltpu.async_copy(ad0_hbm.at[dst_v], drow_v, sem_d)

                @pl.when(c == 1)
                def _():
                    pltpu.async_copy(tabs[2 + gpass].at[src_v], rows_v, sem)
                    pltpu.async_copy(ad1_hbm.at[src_v], srow_v, sem_s)
                    pltpu.async_copy(ad1_hbm.at[dst_v], drow_v, sem_d)

                pltpu.make_async_copy(ad0_hbm.at[src_v], srow_v, sem_s).wait()
                pltpu.make_async_copy(ad0_hbm.at[dst_v], drow_v, sem_d).wait()

                _compute_e(srow_v, drow_v, e_buf, e_srows,
                           [(0, 2 * gpass, 4 + 2 * gpass, 2 * gpass),
                            (1, 2 * gpass + 1, 5 + 2 * gpass, 2 * gpass + 1)])
                pltpu.sync_copy(e_srows, s_sh.at[dst_v], add=True)
                pltpu.make_async_copy(tabs[gpass].at[src_v], rows_v, sem).wait()

                zero16 = jnp.zeros((16,), _i32)
                one16 = jnp.full((16,), 1, _i32)

                @plsc.parallel_loop(0, CHUNK, unroll=4)
                def _(b):
                    bsp = jnp.full((16,), b, _i32)
                    e0 = plsc.load_gather(e_buf, [bsp, zero16])
                    e1 = plsc.load_gather(e_buf, [bsp, one16])
                    for j in range(8):
                        ev = e0 if j < 4 else e1
                        sl = pl.ds(j * 16, 16)
                        rows_v[b, sl] = rows_v[b, sl] * ev

                pltpu.sync_copy(rows_v, acc_sh.at[dst_v], add=True)

            plsc.subcore_barrier()
            for kk in range(rps // CHUNK):
                sl = pl.ds(rbase + kk * CHUNK, CHUNK)

                @pl.when(c == 0)
                def _():
                    pltpu.sync_copy(acc_sh.at[sl], outs[gpass].at[sl])

                @pl.when(c == 1)
                def _():
                    pltpu.sync_copy(acc_sh.at[sl], outs[2 + gpass].at[sl])

            plsc.subcore_barrier()

        for kk in range(rps // CHUNK):
            sl = pl.ds(rbase + kk * CHUNK, CHUNK)

            @pl.when(c == 0)
            def _():
                pltpu.sync_copy(s_sh.at[sl], s0.at[sl])

            @pl.when(c == 1)
            def _():
                pltpu.sync_copy(s_sh.at[sl], s1.at[sl])

    return k(srcp, dstp, t0, t1, t2, t3, ad0, ad1)


def _sc_layer2(srcp, dstp, tA, tB, ad0, ad1):
    rps = N_ACC // NSUB

    @functools.partial(
        pl.kernel,
        out_type=[_sds((N_ACC, 64))] * 2 + [_sds((N_ACC, 16))] * 2,
        mesh=plsc.VectorSubcoreMesh(**_MESH),
        compiler_params=_CP,
        scratch_types=[
            pltpu.VMEM((CHUNK,), _i32),
            pltpu.VMEM((CHUNK,), _i32),
            pltpu.VMEM((CHUNK, 64), _f32),
            pltpu.VMEM((CHUNK, 8), _f32),
            pltpu.VMEM((CHUNK, 8), _f32),
            pltpu.VMEM((CHUNK, 4), _f32),
            pltpu.VMEM((CHUNK, 16), _f32),
            pltpu.VMEM_SHARED((N_ACC, 64), _f32),
            pltpu.VMEM_SHARED((N_ACC, 16), _f32),
            pltpu.SemaphoreType.DMA,
            pltpu.SemaphoreType.DMA,
            pltpu.SemaphoreType.DMA,
        ],
    )
    def k(src_hbm, dst_hbm, ta_hbm, tb_hbm, ad0_hbm, ad1_hbm,
          oA, oB, s0, s1,
          src_v, dst_v, rows_v, srow_v, drow_v, e_buf, e_srows,
          acc_sh, s_sh, sem, sem_s, sem_d):
        c = lax.axis_index("c")
        s = lax.axis_index("s")
        ebase = s * (NCHUNK * CHUNK)
        rbase = s * rps

        _zero_rows(rows_v, CHUNK, 64)
        _zero_rows(e_srows, CHUNK, 16)
        for kk in range(rps // CHUNK):
            sl = pl.ds(rbase + kk * CHUNK, CHUNK)
            pltpu.sync_copy(rows_v, acc_sh.at[sl])
            pltpu.sync_copy(e_srows, s_sh.at[sl])
        plsc.subcore_barrier()

        @pl.loop(0, NCHUNK)
        def _(i):
            base = ebase + i * CHUNK
            pltpu.sync_copy(src_hbm.at[pl.ds(base, CHUNK)], src_v)
            pltpu.sync_copy(dst_hbm.at[pl.ds(base, CHUNK)], dst_v)

            @pl.when(c == 0)
            def _():
                pltpu.async_copy(ta_hbm.at[src_v], rows_v, sem)
                pltpu.async_copy(ad0_hbm.at[src_v], srow_v, sem_s)
                pltpu.async_copy(ad0_hbm.at[dst_v], drow_v, sem_d)

            @pl.when(c == 1)
            def _():
                pltpu.async_copy(tb_hbm.at[src_v], rows_v, sem)
                pltpu.async_copy(ad1_hbm.at[src_v], srow_v, sem_s)
                pltpu.async_copy(ad1_hbm.at[dst_v], drow_v, sem_d)

            pltpu.make_async_copy(ad0_hbm.at[src_v], srow_v, sem_s).wait()
            pltpu.make_async_copy(ad0_hbm.at[dst_v], drow_v, sem_d).wait()

            _compute_e(srow_v, drow_v, e_buf, e_srows,
                       [(j, j, 4 + j, j) for j in range(4)])
            pltpu.sync_copy(e_srows, s_sh.at[dst_v], add=True)
            pltpu.make_async_copy(ta_hbm.at[src_v], rows_v, sem).wait()

            @plsc.parallel_loop(0, CHUNK, unroll=4)
            def _(b):
                bsp = jnp.full((16,), b, _i32)
                for j in range(4):
                    ev = plsc.load_gather(e_buf, [bsp, jnp.full((16,), j, _i32)])
                    sl = pl.ds(j * 16, 16)
                    rows_v[b, sl] = rows_v[b, sl] * ev

            pltpu.sync_copy(rows_v, acc_sh.at[dst_v], add=True)

        plsc.subcore_barrier()
        for kk in range(rps // CHUNK):
            sl = pl.ds(rbase + kk * CHUNK, CHUNK)

            @pl.when(c == 0)
            def _():
                pltpu.sync_copy(acc_sh.at[sl], oA.at[sl])
                pltpu.sync_copy(s_sh.at[sl], s0.at[sl])

            @pl.when(c == 1)
            def _():
                pltpu.sync_copy(acc_sh.at[sl], oB.at[sl])
                pltpu.sync_copy(s_sh.at[sl], s1.at[sl])

    return k(srcp, dstp, tA, tB, ad0, ad1)


def _tc_mid_body(a0, a1, a2, a3, s0_ref, s1_ref, b1_ref, w2_ref,
                 as_ref, ad_ref, tA, tB, ad0_ref, ad1_ref):
    parts = []
    for g in range(4):
        a = (a0, a1, a2, a3)[g][...]
        s_ref = s0_ref if g < 2 else s1_ref
        for d in range(2):
            h = 2 * g + d
            lane = h - 4 * (h // 4)
            den = s_ref[:, lane:lane + 1] + 1e-16
            parts.append(a[:, d * 64:(d + 1) * 64] / den)
    h1 = jnp.concatenate(parts, axis=1) + b1_ref[...]
    h1 = jnp.where(h1 > 0, h1, jnp.exp(h1) - 1.0)
    xp2 = jnp.dot(h1, w2_ref[...], preferred_element_type=_f32)
    tA[...] = xp2[:, :64]
    tB[...] = xp2[:, 64:]
    ad0_ref[...], ad1_ref[...] = _head_scalars(xp2, as_ref[...], ad_ref[...], C2)


def _tc_mid(accs, s0, s1, b1, W2, att_src2, att_dst2):
    return pl.pallas_call(
        _tc_mid_body,
        grid=(N_ACC // BM,),
        in_specs=[pl.BlockSpec((BM, 128), lambda i: (i, 0))] * 4
        + [pl.BlockSpec((BM, 16), lambda i: (i, 0))] * 2
        + [
            pl.BlockSpec((1, H * C1), lambda i: (0, 0)),
            pl.BlockSpec((H * C1, H * C2), lambda i: (0, 0)),
            pl.BlockSpec((H, C2), lambda i: (0, 0)),
            pl.BlockSpec((H, C2), lambda i: (0, 0)),
        ],
        out_specs=[pl.BlockSpec((BM, 64), lambda i: (i, 0))] * 2
        + [pl.BlockSpec((BM, 8), lambda i: (i, 0))] * 2,
        out_shape=[_sds((N_ACC, 64))] * 2 + [_sds((N_ACC, 8))] * 2,
    )(*accs, s0, s1, b1.reshape(1, -1), W2, att_src2, att_dst2)


def _tc_post_body(aA, aB, s0_ref, s1_ref, b2_ref, o_ref):
    parts = []
    for h in range(H):
        acc = (aA if h < 4 else aB)[...]
        s_ref = s0_ref if h < 4 else s1_ref
        j = h % 4
        den = s_ref[:, j:j + 1] + 1e-16
        parts.append(acc[:, j * 16:(j + 1) * 16] / den)
    o_ref[...] = jnp.concatenate(parts, axis=1) + b2_ref[...]


def _tc_post(aA, aB, s0, s1, b2):
    bm = 1000
    return pl.pallas_call(
        _tc_post_body,
        grid=(N // bm,),
        in_specs=[pl.BlockSpec((bm, 64), lambda i: (i, 0))] * 2
        + [pl.BlockSpec((bm, 16), lambda i: (i, 0))] * 2
        + [pl.BlockSpec((1, H * C2), lambda i: (0, 0))],
        out_specs=pl.BlockSpec((bm, H * C2), lambda i: (i, 0)),
        out_shape=_sds((N, H * C2)),
    )(aA, aB, s0, s1, b2.reshape(1, -1))


def kernel(x, edge_index, W1, att_src1, att_dst1, b1, W2, att_src2, att_dst2, b2):
    loop = jnp.arange(N, dtype=jnp.int32)
    pad = jnp.full((EPS - E_TOT,), N, jnp.int32)
    srcp = jnp.concatenate([edge_index[0].astype(jnp.int32), loop, pad])
    dstp = jnp.concatenate([edge_index[1].astype(jnp.int32), loop, pad])
    x_pad = jnp.pad(x, ((0, N_ACC - N), (0, 0)))

    t0, t1, t2, t3, ad1_0, ad1_1 = _tc_pre(x_pad, W1, att_src1, att_dst1)
    a0, a1, a2, a3, s0, s1 = _sc_layer1(srcp, dstp, t0, t1, t2, t3, ad1_0, ad1_1)
    tA, tB, ad2_0, ad2_1 = _tc_mid((a0, a1, a2, a3), s0, s1, b1, W2,
                                   att_src2, att_dst2)
    aA, aB, s20, s21 = _sc_layer2(srcp, dstp, tA, tB, ad2_0, ad2_1)
    return _tc_post(aA, aB, s20, s21, b2)


# speedup vs baseline: 40.0704x; 1.0439x over previous
import dataclasses
import functools

import jax
import jax.numpy as jnp
from jax import lax
from jax.experimental import pallas as pl
from jax.experimental.pallas import tpu as pltpu
from jax.experimental.pallas import tpu_sc as plsc

N = 10000
E = 320000
F_IN = 128
H = 8
C1 = 64
C2 = 16

N_ACC = 10240
BM = 1024
CHUNK = 128
HC = 64
NSUB = 16
NCHUNK = 162
EPS = NSUB * NCHUNK * CHUNK
E_TOT = E + N

_MESH = dict(core_axis_name="c", subcore_axis_name="s")

_CP = pltpu.CompilerParams()
if "needs_layout_passes" in pltpu.CompilerParams.__dataclass_fields__:
    _CP = dataclasses.replace(_CP, needs_layout_passes=False)
if "use_tc_tiling_on_sc" in pltpu.CompilerParams.__dataclass_fields__:
    _CP = dataclasses.replace(_CP, use_tc_tiling_on_sc=False)

_f32 = jnp.float32
_i32 = jnp.int32


def _sds(shape, dtype=_f32):
    return jax.ShapeDtypeStruct(shape, dtype)


def _head_scalars(xp, att_src, att_dst, ch):
    cols_s, cols_d = [], []
    for h in range(H):
        xh = xp[:, h * ch:(h + 1) * ch]
        cols_s.append(jnp.sum(xh * att_src[h][None, :], axis=1, keepdims=True))
        cols_d.append(jnp.sum(xh * att_dst[h][None, :], axis=1, keepdims=True))
    asrc = jnp.concatenate(cols_s, axis=1)
    adst = jnp.concatenate(cols_d, axis=1)
    ad0 = jnp.concatenate([asrc[:, :4], adst[:, :4]], axis=1)
    ad1 = jnp.concatenate([asrc[:, 4:], adst[:, 4:]], axis=1)
    return ad0, ad1


def _tc_pre_body(x_ref, w1_ref, as_ref, ad_ref,
                 t0, t1, t2, t3, ad0_ref, ad1_ref):
    xp = jnp.dot(x_ref[...], w1_ref[...], preferred_element_type=_f32)
    for g, t in enumerate((t0, t1, t2, t3)):
        t[...] = xp[:, g * 128:(g + 1) * 128]
    ad0_ref[...], ad1_ref[...] = _head_scalars(xp, as_ref[...], ad_ref[...], C1)


def _tc_pre(x_pad, W1, att_src1, att_dst1):
    return pl.pallas_call(
        _tc_pre_body,
        grid=(N_ACC // BM,),
        in_specs=[
            pl.BlockSpec((BM, F_IN), lambda i: (i, 0)),
            pl.BlockSpec((F_IN, H * C1), lambda i: (0, 0)),
            pl.BlockSpec((H, C1), lambda i: (0, 0)),
            pl.BlockSpec((H, C1), lambda i: (0, 0)),
        ],
        out_specs=[pl.BlockSpec((BM, 128), lambda i: (i, 0))] * 4
        + [pl.BlockSpec((BM, 8), lambda i: (i, 0))] * 2,
        out_shape=[_sds((N_ACC, 128))] * 4 + [_sds((N_ACC, 8))] * 2,
    )(x_pad, W1, att_src1, att_dst1)


def _zero_rows(ref, nrow, ncol):
    z = jnp.zeros((16,), _f32)

    @pl.loop(0, nrow)
    def _(b):
        for j in range(ncol // 16):
            ref[b, pl.ds(j * 16, 16)] = z


def _compute_e(srow_v, drow_v, e_buf, e_srows, entries):
    iota = lax.iota(_i32, 16)

    @plsc.parallel_loop(0, HC // 16, unroll=2)
    def _(k):
        idx16 = iota + k * 16
        for (jc, sl, dl, ol) in entries:
            av = plsc.load_gather(srow_v, [idx16, jnp.full((16,), sl, _i32)])
            bv = plsc.load_gather(drow_v, [idx16, jnp.full((16,), dl, _i32)])
            l = av + bv
            e16 = jnp.exp(jnp.maximum(l, l * 0.2))
            plsc.store_scatter(e_buf, [idx16, jnp.full((16,), jc, _i32)], e16)
            plsc.store_scatter(e_srows, [idx16, jnp.full((16,), ol, _i32)], e16)


def _edge_pipeline(c, s, tab0, tab1, ad0, ad1, src_hbm, dst_hbm,
                   src_v, dst_v, rows_v, srow_v, drow_v, e_buf, e_srows,
                   acc_sh, s_sh, sem_r, sem_s, sem_d, entries, D):
    ebase = s * (NCHUNK * CHUNK)
    ncol_e = len(entries)
    nj = D // 16

    def prefetch(hidx, st):
        base = ebase + hidx * HC
        pltpu.sync_copy(src_hbm.at[pl.ds(base, HC)], src_v.at[st])
        pltpu.sync_copy(dst_hbm.at[pl.ds(base, HC)], dst_v.at[st])

        @pl.when(c == 0)
        def _():
            pltpu.async_copy(tab0.at[src_v.at[st]], rows_v.at[st], sem_r[st])
            pltpu.async_copy(ad0.at[src_v.at[st]], srow_v.at[st], sem_s[st])
            pltpu.async_copy(ad0.at[dst_v.at[st]], drow_v.at[st], sem_d[st])

        @pl.when(c == 1)
        def _():
            pltpu.async_copy(tab1.at[src_v.at[st]], rows_v.at[st], sem_r[st])
            pltpu.async_copy(ad1.at[src_v.at[st]], srow_v.at[st], sem_s[st])
            pltpu.async_copy(ad1.at[dst_v.at[st]], drow_v.at[st], sem_d[st])

    def process(st):
        pltpu.make_async_copy(ad0.at[src_v.at[st]], srow_v.at[st], sem_s[st]).wait()
        pltpu.make_async_copy(ad0.at[dst_v.at[st]], drow_v.at[st], sem_d[st]).wait()
        _compute_e(srow_v.at[st], drow_v.at[st], e_buf.at[st], e_srows.at[st],
                   entries)
        pltpu.sync_copy(e_srows.at[st], s_sh.at[dst_v.at[st]], add=True)
        pltpu.make_async_copy(tab0.at[src_v.at[st]], rows_v.at[st], sem_r[st]).wait()

        @plsc.parallel_loop(0, HC, unroll=4)
        def _(b):
            bsp = jnp.full((16,), b, _i32)
            evs = [plsc.load_gather(e_buf.at[st], [bsp, jnp.full((16,), jc, _i32)])
                   for jc in range(ncol_e)]
            R = rows_v.at[st]
            for j in range(nj):
                ev = evs[j // (nj // ncol_e)]
                sl = pl.ds(j * 16, 16)
                R[b, sl] = R[b, sl] * ev

        pltpu.sync_copy(rows_v.at[st], acc_sh.at[dst_v.at[st]], add=True)

    prefetch(0, 0)

    @pl.loop(0, NCHUNK)
    def _(i):
        prefetch(2 * i + 1, 1)
        process(0)

        @pl.when(i < NCHUNK - 1)
        def _():
            prefetch(2 * i + 2, 0)

        process(1)


def _sc_layer1(srcp, dstp, t0, t1, t2, t3, ad0, ad1):
    rps = N_ACC // NSUB

    @functools.partial(
        pl.kernel,
        out_type=[_sds((N_ACC, 128))] * 4 + [_sds((N_ACC, 16))] * 2,
        mesh=plsc.VectorSubcoreMesh(**_MESH),
        compiler_params=_CP,
        scratch_types=[
            pltpu.VMEM((2, HC), _i32),
            pltpu.VMEM((2, HC), _i32),
            pltpu.VMEM((2, HC, 128), _f32),
            pltpu.VMEM((2, HC, 8), _f32),
            pltpu.VMEM((2, HC, 8), _f32),
            pltpu.VMEM((2, HC, 2), _f32),
            pltpu.VMEM((2, HC, 16), _f32),
            pltpu.VMEM_SHARED((N_ACC, 128), _f32),
            pltpu.VMEM_SHARED((N_ACC, 16), _f32),
            pltpu.SemaphoreType.DMA,
            pltpu.SemaphoreType.DMA,
            pltpu.SemaphoreType.DMA,
            pltpu.SemaphoreType.DMA,
            pltpu.SemaphoreType.DMA,
            pltpu.SemaphoreType.DMA,
        ],
    )
    def k(src_hbm, dst_hbm, t0_hbm, t1_hbm, t2_hbm, t3_hbm, ad0_hbm, ad1_hbm,
          o0, o1, o2, o3, s0, s1,
          src_v, dst_v, rows_v, srow_v, drow_v, e_buf, e_srows,
          acc_sh, s_sh, sr0, sr1, ss0, ss1, sd0, sd1):
        c = lax.axis_index("c")
        s = lax.axis_index("s")
        rbase = s * rps
        outs = (o0, o1, o2, o3)
        tabs = (t0_hbm, t1_hbm, t2_hbm, t3_hbm)

        for gpass in range(2):
            for st in range(2):
                _zero_rows(rows_v.at[st], HC, 128)
                _zero_rows(e_srows.at[st], HC, 16)
            for kk in range(rps // HC):
                sl = pl.ds(rbase + kk * HC, HC)
                pltpu.sync_copy(rows_v.at[0], acc_sh.at[sl])
                if gpass == 0:
                    pltpu.sync_copy(e_srows.at[0], s_sh.at[sl])
            plsc.subcore_barrier()

            _edge_pipeline(
                c, s, tabs[gpass], tabs[2 + gpass], ad0_hbm, ad1_hbm,
                src_hbm, dst_hbm, src_v, dst_v, rows_v, srow_v, drow_v,
                e_buf, e_srows, acc_sh, s_sh,
                (sr0, sr1), (ss0, ss1), (sd0, sd1),
                [(0, 2 * gpass, 4 + 2 * gpass, 2 * gpass),
                 (1, 2 * gpass + 1, 5 + 2 * gpass, 2 * gpass + 1)],
                128)

            plsc.subcore_barrier()
            for kk in range(rps // HC):
                sl = pl.ds(rbase + kk * HC, HC)

                @pl.when(c == 0)
                def _():
                    pltpu.sync_copy(acc_sh.at[sl], outs[gpass].at[sl])

                @pl.when(c == 1)
                def _():
                    pltpu.sync_copy(acc_sh.at[sl], outs[2 + gpass].at[sl])

            plsc.subcore_barrier()

        for kk in range(rps // HC):
            sl = pl.ds(rbase + kk * HC, HC)

            @pl.when(c == 0)
            def _():
                pltpu.sync_copy(s_sh.at[sl], s0.at[sl])

            @pl.when(c == 1)
            def _():
                pltpu.sync_copy(s_sh.at[sl], s1.at[sl])

    return k(srcp, dstp, t0, t1, t2, t3, ad0, ad1)


def _sc_layer2(srcp, dstp, tA, tB, ad0, ad1):
    rps = N_ACC // NSUB

    @functools.partial(
        pl.kernel,
        out_type=[_sds((N_ACC, 64))] * 2 + [_sds((N_ACC, 16))] * 2,
        mesh=plsc.VectorSubcoreMesh(**_MESH),
        compiler_params=_CP,
        scratch_types=[
            pltpu.VMEM((2, HC), _i32),
            pltpu.VMEM((2, HC), _i32),
            pltpu.VMEM((2, HC, 64), _f32),
            pltpu.VMEM((2, HC, 8), _f32),
            pltpu.VMEM((2, HC, 8), _f32),
            pltpu.VMEM((2, HC, 4), _f32),
            pltpu.VMEM((2, HC, 16), _f32),
            pltpu.VMEM_SHARED((N_ACC, 64), _f32),
            pltpu.VMEM_SHARED((N_ACC, 16), _f32),
            pltpu.SemaphoreType.DMA,
            pltpu.SemaphoreType.DMA,
            pltpu.SemaphoreType.DMA,
            pltpu.SemaphoreType.DMA,
            pltpu.SemaphoreType.DMA,
            pltpu.SemaphoreType.DMA,
        ],
    )
    def k(src_hbm, dst_hbm, ta_hbm, tb_hbm, ad0_hbm, ad1_hbm,
          oA, oB, s0, s1,
          src_v, dst_v, rows_v, srow_v, drow_v, e_buf, e_srows,
          acc_sh, s_sh, sr0, sr1, ss0, ss1, sd0, sd1):
        c = lax.axis_index("c")
        s = lax.axis_index("s")
        rbase = s * rps

        for st in range(2):
            _zero_rows(rows_v.at[st], HC, 64)
            _zero_rows(e_srows.at[st], HC, 16)
        for kk in range(rps // HC):
            sl = pl.ds(rbase + kk * HC, HC)
            pltpu.sync_copy(rows_v.at[0], acc_sh.at[sl])
            pltpu.sync_copy(e_srows.at[0], s_sh.at[sl])
        plsc.subcore_barrier()

        _edge_pipeline(
            c, s, ta_hbm, tb_hbm, ad0_hbm, ad1_hbm,
            src_hbm, dst_hbm, src_v, dst_v, rows_v, srow_v, drow_v,
            e_buf, e_srows, acc_sh, s_sh,
            (sr0, sr1), (ss0, ss1), (sd0, sd1),
            [(j, j, 4 + j, j) for j in range(4)],
            64)

        plsc.subcore_barrier()
        for kk in range(rps // HC):
            sl = pl.ds(rbase + kk * HC, HC)

            @pl.when(c == 0)
            def _():
                pltpu.sync_copy(acc_sh.at[sl], oA.at[sl])
                pltpu.sync_copy(s_sh.at[sl], s0.at[sl])

            @pl.when(c == 1)
            def _():
                pltpu.sync_copy(acc_sh.at[sl], oB.at[sl])
                pltpu.sync_copy(s_sh.at[sl], s1.at[sl])

    return k(srcp, dstp, tA, tB, ad0, ad1)


def _tc_mid_body(a0, a1, a2, a3, s0_ref, s1_ref, b1_ref, w2_ref,
                 as_ref, ad_ref, tA, tB, ad0_ref, ad1_ref):
    parts = []
    for g in range(4):
        a = (a0, a1, a2, a3)[g][...]
        s_ref = s0_ref if g < 2 else s1_ref
        for d in range(2):
            h = 2 * g + d
            lane = h - 4 * (h // 4)
            den = s_ref[:, lane:lane + 1] + 1e-16
            parts.append(a[:, d * 64:(d + 1) * 64] / den)
    h1 = jnp.concatenate(parts, axis=1) + b1_ref[...]
    h1 = jnp.where(h1 > 0, h1, jnp.exp(h1) - 1.0)
    xp2 = jnp.dot(h1, w2_ref[...], preferred_element_type=_f32)
    tA[...] = xp2[:, :64]
    tB[...] = xp2[:, 64:]
    ad0_ref[...], ad1_ref[...] = _head_scalars(xp2, as_ref[...], ad_ref[...], C2)


def _tc_mid(accs, s0, s1, b1, W2, att_src2, att_dst2):
    return pl.pallas_call(
        _tc_mid_body,
        grid=(N_ACC // BM,),
        in_specs=[pl.BlockSpec((BM, 128), lambda i: (i, 0))] * 4
        + [pl.BlockSpec((BM, 16), lambda i: (i, 0))] * 2
        + [
            pl.BlockSpec((1, H * C1), lambda i: (0, 0)),
            pl.BlockSpec((H * C1, H * C2), lambda i: (0, 0)),
            pl.BlockSpec((H, C2), lambda i: (0, 0)),
            pl.BlockSpec((H, C2), lambda i: (0, 0)),
        ],
        out_specs=[pl.BlockSpec((BM, 64), lambda i: (i, 0))] * 2
        + [pl.BlockSpec((BM, 8), lambda i: (i, 0))] * 2,
        out_shape=[_sds((N_ACC, 64))] * 2 + [_sds((N_ACC, 8))] * 2,
    )(*accs, s0, s1, b1.reshape(1, -1), W2, att_src2, att_dst2)


def _tc_post_body(aA, aB, s0_ref, s1_ref, b2_ref, o_ref):
    parts = []
    for h in range(H):
        acc = (aA if h < 4 else aB)[...]
        s_ref = s0_ref if h < 4 else s1_ref
        j = h % 4
        den = s_ref[:, j:j + 1] + 1e-16
        parts.append(acc[:, j * 16:(j + 1) * 16] / den)
    o_ref[...] = jnp.concatenate(parts, axis=1) + b2_ref[...]


def _tc_post(aA, aB, s0, s1, b2):
    bm = 1000
    return pl.pallas_call(
        _tc_post_body,
        grid=(N // bm,),
        in_specs=[pl.BlockSpec((bm, 64), lambda i: (i, 0))] * 2
        + [pl.BlockSpec((bm, 16), lambda i: (i, 0))] * 2
        + [pl.BlockSpec((1, H * C2), lambda i: (0, 0))],
        out_specs=pl.BlockSpec((bm, H * C2), lambda i: (i, 0)),
        out_shape=_sds((N, H * C2)),
    )(aA, aB, s0, s1, b2.reshape(1, -1))


def kernel(x, edge_index, W1, att_src1, att_dst1, b1, W2, att_src2, att_dst2, b2):
    loop = jnp.arange(N, dtype=jnp.int32)
    pad = jnp.full((EPS - E_TOT,), N, jnp.int32)
    srcp = jnp.concatenate([edge_index[0].astype(jnp.int32), loop, pad])
    dstp = jnp.concatenate([edge_index[1].astype(jnp.int32), loop, pad])
    x_pad = jnp.pad(x, ((0, N_ACC - N), (0, 0)))

    t0, t1, t2, t3, ad1_0, ad1_1 = _tc_pre(x_pad, W1, att_src1, att_dst1)
    a0, a1, a2, a3, s0, s1 = _sc_layer1(srcp, dstp, t0, t1, t2, t3, ad1_0, ad1_1)
    tA, tB, ad2_0, ad2_1 = _tc_mid((a0, a1, a2, a3), s0, s1, b1, W2,
                                   att_src2, att_dst2)
    aA, aB, s20, s21 = _sc_layer2(srcp, dstp, tA, tB, ad2_0, ad2_1)
    return _tc_post(aA, aB, s20, s21, b2)


# speedup vs baseline: 49.6936x; 1.2402x over previous
import dataclasses
import functools

import jax
import jax.numpy as jnp
from jax import lax
from jax.experimental import pallas as pl
from jax.experimental.pallas import tpu as pltpu
from jax.experimental.pallas import tpu_sc as plsc

N = 10000
E = 320000
F_IN = 128
H = 8
C1 = 64
C2 = 16

N_ACC = 10240
BM = 1024
CHUNK = 128
HC = 96
NST = 2
NSUB = 16
NCHUNK = 162
EPS = NSUB * NCHUNK * CHUNK
E_TOT = E + N

_MESH = dict(core_axis_name="c", subcore_axis_name="s")

_CP = pltpu.CompilerParams()
if "needs_layout_passes" in pltpu.CompilerParams.__dataclass_fields__:
    _CP = dataclasses.replace(_CP, needs_layout_passes=False)
if "use_tc_tiling_on_sc" in pltpu.CompilerParams.__dataclass_fields__:
    _CP = dataclasses.replace(_CP, use_tc_tiling_on_sc=False)

_f32 = jnp.float32
_i32 = jnp.int32


def _sds(shape, dtype=_f32):
    return jax.ShapeDtypeStruct(shape, dtype)


def _head_scalars(xp, att_src, att_dst, ch):
    cols_s, cols_d = [], []
    for h in range(H):
        xh = xp[:, h * ch:(h + 1) * ch]
        cols_s.append(jnp.sum(xh * att_src[h][None, :], axis=1, keepdims=True))
        cols_d.append(jnp.sum(xh * att_dst[h][None, :], axis=1, keepdims=True))
    asrc = jnp.concatenate(cols_s, axis=1)
    adst = jnp.concatenate(cols_d, axis=1)
    ad0 = jnp.concatenate([asrc[:, :4], adst[:, :4]], axis=1)
    ad1 = jnp.concatenate([asrc[:, 4:], adst[:, 4:]], axis=1)
    return ad0, ad1


def _tc_pre_body(x_ref, w1_ref, as_ref, ad_ref,
                 t0, t1, t2, t3, ad0_ref, ad1_ref):
    xp = jnp.dot(x_ref[...], w1_ref[...], preferred_element_type=_f32)
    for g, t in enumerate((t0, t1, t2, t3)):
        t[...] = xp[:, g * 128:(g + 1) * 128]
    ad0_ref[...], ad1_ref[...] = _head_scalars(xp, as_ref[...], ad_ref[...], C1)


def _tc_pre(x_pad, W1, att_src1, att_dst1):
    return pl.pallas_call(
        _tc_pre_body,
        grid=(N_ACC // BM,),
        in_specs=[
            pl.BlockSpec((BM, F_IN), lambda i: (i, 0)),
            pl.BlockSpec((F_IN, H * C1), lambda i: (0, 0)),
            pl.BlockSpec((H, C1), lambda i: (0, 0)),
            pl.BlockSpec((H, C1), lambda i: (0, 0)),
        ],
        out_specs=[pl.BlockSpec((BM, 128), lambda i: (i, 0))] * 4
        + [pl.BlockSpec((BM, 8), lambda i: (i, 0))] * 2,
        out_shape=[_sds((N_ACC, 128))] * 4 + [_sds((N_ACC, 8))] * 2,
    )(x_pad, W1, att_src1, att_dst1)


def _zero_rows(ref, nrow, ncol):
    z = jnp.zeros((16,), _f32)

    @pl.loop(0, nrow)
    def _(b):
        for j in range(ncol // 16):
            ref[b, pl.ds(j * 16, 16)] = z


def _compute_e(srow_v, drow_v, e_buf, e_srows, entries):
    iota = lax.iota(_i32, 16)

    @plsc.parallel_loop(0, HC // 16, unroll=2)
    def _(k):
        idx16 = iota + k * 16
        for (jc, sl, dl, ol) in entries:
            av = plsc.load_gather(srow_v, [idx16, jnp.full((16,), sl, _i32)])
            bv = plsc.load_gather(drow_v, [idx16, jnp.full((16,), dl, _i32)])
            l = av + bv
            e16 = jnp.exp(jnp.maximum(l, l * 0.2))
            plsc.store_scatter(e_buf, [idx16, jnp.full((16,), jc, _i32)], e16)
            plsc.store_scatter(e_srows, [idx16, jnp.full((16,), ol, _i32)], e16)


def _edge_pipeline(c, s, tab0, tab1, ad0, ad1, src_hbm, dst_hbm,
                   src_v, dst_v, rows_v, srow_v, drow_v, e_buf, e_srows,
                   acc_sh, s_sh, sem_r, sem_s, sem_d, sem_e, sem_a,
                   entries, D):
    ebase = s * (NCHUNK * CHUNK)
    ncol_e = len(entries)
    nj = D // 16
    TH = (NCHUNK * CHUNK) // HC

    def drain(st):
        pltpu.make_async_copy(e_srows.at[st], s_sh.at[dst_v.at[st]],
                              sem_e[st]).wait()
        pltpu.make_async_copy(rows_v.at[st], acc_sh.at[dst_v.at[st]],
                              sem_a[st]).wait()

    def prefetch(hidx, st, guard_drain):
        if guard_drain:
            @pl.when(hidx >= NST)
            def _():
                drain(st)
        base = ebase + hidx * HC
        pltpu.sync_copy(src_hbm.at[pl.ds(base, HC)], src_v.at[st])
        pltpu.sync_copy(dst_hbm.at[pl.ds(base, HC)], dst_v.at[st])

        @pl.when(c == 0)
        def _():
            pltpu.async_copy(tab0.at[src_v.at[st]], rows_v.at[st], sem_r[st])
            pltpu.async_copy(ad0.at[src_v.at[st]], srow_v.at[st], sem_s[st])
            pltpu.async_copy(ad0.at[dst_v.at[st]], drow_v.at[st], sem_d[st])

        @pl.when(c == 1)
        def _():
            pltpu.async_copy(tab1.at[src_v.at[st]], rows_v.at[st], sem_r[st])
            pltpu.async_copy(ad1.at[src_v.at[st]], srow_v.at[st], sem_s[st])
            pltpu.async_copy(ad1.at[dst_v.at[st]], drow_v.at[st], sem_d[st])

    def process(st):
        pltpu.make_async_copy(ad0.at[src_v.at[st]], srow_v.at[st], sem_s[st]).wait()
        pltpu.make_async_copy(ad0.at[dst_v.at[st]], drow_v.at[st], sem_d[st]).wait()
        _compute_e(srow_v.at[st], drow_v.at[st], e_buf.at[st], e_srows.at[st],
                   entries)
        pltpu.async_copy(e_srows.at[st], s_sh.at[dst_v.at[st]], sem_e[st],
                         add=True)
        pltpu.make_async_copy(tab0.at[src_v.at[st]], rows_v.at[st], sem_r[st]).wait()

        @plsc.parallel_loop(0, HC, unroll=4)
        def _(b):
            bsp = jnp.full((16,), b, _i32)
            evs = [plsc.load_gather(e_buf.at[st], [bsp, jnp.full((16,), jc, _i32)])
                   for jc in range(ncol_e)]
            R = rows_v.at[st]
            for j in range(nj):
                ev = evs[j // (nj // ncol_e)]
                sl = pl.ds(j * 16, 16)
                R[b, sl] = R[b, sl] * ev

        pltpu.async_copy(rows_v.at[st], acc_sh.at[dst_v.at[st]], sem_a[st],
                         add=True)

    for p in range(NST - 1):
        prefetch(p, p, False)

    @pl.loop(0, TH // NST)
    def _(i):
        h0 = NST * i
        for d in range(NST):
            hp = h0 + d + NST - 1
            stp = (d + NST - 1) % NST
            if d == 0:
                prefetch(hp, stp, True)
            else:
                def _pref(hp=hp, stp=stp):
                    @pl.when(hp < TH)
                    def _():
                        prefetch(hp, stp, True)
                _pref()
            process(d)

    for st in range(NST):
        drain(st)


def _sc_layer1(srcp, dstp, t0, t1, t2, t3, ad0, ad1):
    rps = N_ACC // NSUB

    @functools.partial(
        pl.kernel,
        out_type=[_sds((N_ACC, 128))] * 4 + [_sds((N_ACC, 16))] * 2,
        mesh=plsc.VectorSubcoreMesh(**_MESH),
        compiler_params=_CP,
        scratch_types=[
            pltpu.VMEM((NST, HC), _i32),
            pltpu.VMEM((NST, HC), _i32),
            pltpu.VMEM((NST, HC, 128), _f32),
            pltpu.VMEM((NST, HC, 8), _f32),
            pltpu.VMEM((NST, HC, 8), _f32),
            pltpu.VMEM((NST, HC, 2), _f32),
            pltpu.VMEM((NST, HC, 16), _f32),
            pltpu.VMEM_SHARED((N_ACC, 128), _f32),
            pltpu.VMEM_SHARED((N_ACC, 16), _f32),
        ] + [pltpu.SemaphoreType.DMA] * (5 * NST),
    )
    def k(src_hbm, dst_hbm, t0_hbm, t1_hbm, t2_hbm, t3_hbm, ad0_hbm, ad1_hbm,
          o0, o1, o2, o3, s0, s1,
          src_v, dst_v, rows_v, srow_v, drow_v, e_buf, e_srows,
          acc_sh, s_sh, *sems):
        c = lax.axis_index("c")
        s = lax.axis_index("s")
        rbase = s * rps
        outs = (o0, o1, o2, o3)
        tabs = (t0_hbm, t1_hbm, t2_hbm, t3_hbm)

        for gpass in range(2):
            for st in range(NST):
                _zero_rows(rows_v.at[st], HC, 128)
                _zero_rows(e_srows.at[st], HC, 16)
            for kk in range(rps // HC):
                sl = pl.ds(rbase + kk * HC, HC)
                pltpu.sync_copy(rows_v.at[0], acc_sh.at[sl])
                if gpass == 0:
                    pltpu.sync_copy(e_srows.at[0], s_sh.at[sl])
            rem = rps % HC
            if rem:
                sl = pl.ds(rbase + (rps // HC) * HC, rem)
                pltpu.sync_copy(rows_v.at[0].at[pl.ds(0, rem)], acc_sh.at[sl])
                if gpass == 0:
                    pltpu.sync_copy(e_srows.at[0].at[pl.ds(0, rem)],
                                    s_sh.at[sl])
            plsc.subcore_barrier()

            _edge_pipeline(
                c, s, tabs[gpass], tabs[2 + gpass], ad0_hbm, ad1_hbm,
                src_hbm, dst_hbm, src_v, dst_v, rows_v, srow_v, drow_v,
                e_buf, e_srows, acc_sh, s_sh,
                sems[0:NST], sems[NST:2 * NST], sems[2 * NST:3 * NST],
                sems[3 * NST:4 * NST], sems[4 * NST:5 * NST],
                [(0, 2 * gpass, 4 + 2 * gpass, 2 * gpass),
                 (1, 2 * gpass + 1, 5 + 2 * gpass, 2 * gpass + 1)],
                128)

            plsc.subcore_barrier()
            nstr = rps // HC + (1 if rps % HC else 0)
            for kk in range(nstr):
                w = min(HC, rps - kk * HC)
                sl = pl.ds(rbase + kk * HC, w)

                @pl.when(c == 0)
                def _():
                    pltpu.sync_copy(acc_sh.at[sl], outs[gpass].at[sl])

                @pl.when(c == 1)
                def _():
                    pltpu.sync_copy(acc_sh.at[sl], outs[2 + gpass].at[sl])

            plsc.subcore_barrier()

        nstr = rps // HC + (1 if rps % HC else 0)
        for kk in range(nstr):
            w = min(HC, rps - kk * HC)
            sl = pl.ds(rbase + kk * HC, w)

            @pl.when(c == 0)
            def _():
                pltpu.sync_copy(s_sh.at[sl], s0.at[sl])

            @pl.when(c == 1)
            def _():
                pltpu.sync_copy(s_sh.at[sl], s1.at[sl])

    return k(srcp, dstp, t0, t1, t2, t3, ad0, ad1)


def _sc_layer2(srcp, dstp, tA, tB, ad0, ad1):
    rps = N_ACC // NSUB

    @functools.partial(
        pl.kernel,
        out_type=[_sds((N_ACC, 64))] * 2 + [_sds((N_ACC, 16))] * 2,
        mesh=plsc.VectorSubcoreMesh(**_MESH),
        compiler_params=_CP,
        scratch_types=[
            pltpu.VMEM((NST, HC), _i32),
            pltpu.VMEM((NST, HC), _i32),
            pltpu.VMEM((NST, HC, 64), _f32),
            pltpu.VMEM((NST, HC, 8), _f32),
            pltpu.VMEM((NST, HC, 8), _f32),
            pltpu.VMEM((NST, HC, 4), _f32),
            pltpu.VMEM((NST, HC, 16), _f32),
            pltpu.VMEM_SHARED((N_ACC, 64), _f32),
            pltpu.VMEM_SHARED((N_ACC, 16), _f32),
        ] + [pltpu.SemaphoreType.DMA] * (5 * NST),
    )
    def k(src_hbm, dst_hbm, ta_hbm, tb_hbm, ad0_hbm, ad1_hbm,
          oA, oB, s0, s1,
          src_v, dst_v, rows_v, srow_v, drow_v, e_buf, e_srows,
          acc_sh, s_sh, *sems):
        c = lax.axis_index("c")
        s = lax.axis_index("s")
        rbase = s * rps

        for st in range(NST):
            _zero_rows(rows_v.at[st], HC, 64)
            _zero_rows(e_srows.at[st], HC, 16)
        for kk in range(rps // HC):
            sl = pl.ds(rbase + kk * HC, HC)
            pltpu.sync_copy(rows_v.at[0], acc_sh.at[sl])
            pltpu.sync_copy(e_srows.at[0], s_sh.at[sl])
        rem = rps % HC
        if rem:
            sl = pl.ds(rbase + (rps // HC) * HC, rem)
            pltpu.sync_copy(rows_v.at[0].at[pl.ds(0, rem)], acc_sh.at[sl])
            pltpu.sync_copy(e_srows.at[0].at[pl.ds(0, rem)], s_sh.at[sl])
        plsc.subcore_barrier()

        _edge_pipeline(
            c, s, ta_hbm, tb_hbm, ad0_hbm, ad1_hbm,
            src_hbm, dst_hbm, src_v, dst_v, rows_v, srow_v, drow_v,
            e_buf, e_srows, acc_sh, s_sh,
            sems[0:NST], sems[NST:2 * NST], sems[2 * NST:3 * NST],
            sems[3 * NST:4 * NST], sems[4 * NST:5 * NST],
            [(j, j, 4 + j, j) for j in range(4)],
            64)

        plsc.subcore_barrier()
        nstr = rps // HC + (1 if rps % HC else 0)
        for kk in range(nstr):
            w = min(HC, rps - kk * HC)
            sl = pl.ds(rbase + kk * HC, w)

            @pl.when(c == 0)
            def _():
                pltpu.sync_copy(acc_sh.at[sl], oA.at[sl])
                pltpu.sync_copy(s_sh.at[sl], s0.at[sl])

            @pl.when(c == 1)
            def _():
                pltpu.sync_copy(acc_sh.at[sl], oB.at[sl])
                pltpu.sync_copy(s_sh.at[sl], s1.at[sl])

    return k(srcp, dstp, tA, tB, ad0, ad1)


def _tc_mid_body(a0, a1, a2, a3, s0_ref, s1_ref, b1_ref, w2_ref,
                 as_ref, ad_ref, tA, tB, ad0_ref, ad1_ref):
    parts = []
    for g in range(4):
        a = (a0, a1, a2, a3)[g][...]
        s_ref = s0_ref if g < 2 else s1_ref
        for d in range(2):
            h = 2 * g + d
            lane = h - 4 * (h // 4)
            den = s_ref[:, lane:lane + 1] + 1e-16
            parts.append(a[:, d * 64:(d + 1) * 64] / den)
    h1 = jnp.concatenate(parts, axis=1) + b1_ref[...]
    h1 = jnp.where(h1 > 0, h1, jnp.exp(h1) - 1.0)
    xp2 = jnp.dot(h1, w2_ref[...], preferred_element_type=_f32)
    tA[...] = xp2[:, :64]
    tB[...] = xp2[:, 64:]
    ad0_ref[...], ad1_ref[...] = _head_scalars(xp2, as_ref[...], ad_ref[...], C2)


def _tc_mid(accs, s0, s1, b1, W2, att_src2, att_dst2):
    return pl.pallas_call(
        _tc_mid_body,
        grid=(N_ACC // BM,),
        in_specs=[pl.BlockSpec((BM, 128), lambda i: (i, 0))] * 4
        + [pl.BlockSpec((BM, 16), lambda i: (i, 0))] * 2
        + [
            pl.BlockSpec((1, H * C1), lambda i: (0, 0)),
            pl.BlockSpec((H * C1, H * C2), lambda i: (0, 0)),
            pl.BlockSpec((H, C2), lambda i: (0, 0)),
            pl.BlockSpec((H, C2), lambda i: (0, 0)),
        ],
        out_specs=[pl.BlockSpec((BM, 64), lambda i: (i, 0))] * 2
        + [pl.BlockSpec((BM, 8), lambda i: (i, 0))] * 2,
        out_shape=[_sds((N_ACC, 64))] * 2 + [_sds((N_ACC, 8))] * 2,
    )(*accs, s0, s1, b1.reshape(1, -1), W2, att_src2, att_dst2)


def _tc_post_body(aA, aB, s0_ref, s1_ref, b2_ref, o_ref):
    parts = []
    for h in range(H):
        acc = (aA if h < 4 else aB)[...]
        s_ref = s0_ref if h < 4 else s1_ref
        j = h % 4
        den = s_ref[:, j:j + 1] + 1e-16
        parts.append(acc[:, j * 16:(j + 1) * 16] / den)
    o_ref[...] = jnp.concatenate(parts, axis=1) + b2_ref[...]


def _tc_post(aA, aB, s0, s1, b2):
    bm = 1000
    return pl.pallas_call(
        _tc_post_body,
        grid=(N // bm,),
        in_specs=[pl.BlockSpec((bm, 64), lambda i: (i, 0))] * 2
        + [pl.BlockSpec((bm, 16), lambda i: (i, 0))] * 2
        + [pl.BlockSpec((1, H * C2), lambda i: (0, 0))],
        out_specs=pl.BlockSpec((bm, H * C2), lambda i: (i, 0)),
        out_shape=_sds((N, H * C2)),
    )(aA, aB, s0, s1, b2.reshape(1, -1))


def kernel(x, edge_index, W1, att_src1, att_dst1, b1, W2, att_src2, att_dst2, b2):
    loop = jnp.arange(N, dtype=jnp.int32)
    pad = jnp.full((EPS - E_TOT,), N, jnp.int32)
    srcp = jnp.concatenate([edge_index[0].astype(jnp.int32), loop, pad])
    dstp = jnp.concatenate([edge_index[1].astype(jnp.int32), loop, pad])
    x_pad = jnp.pad(x, ((0, N_ACC - N), (0, 0)))

    t0, t1, t2, t3, ad1_0, ad1_1 = _tc_pre(x_pad, W1, att_src1, att_dst1)
    a0, a1, a2, a3, s0, s1 = _sc_layer1(srcp, dstp, t0, t1, t2, t3, ad1_0, ad1_1)
    tA, tB, ad2_0, ad2_1 = _tc_mid((a0, a1, a2, a3), s0, s1, b1, W2,
                                   att_src2, att_dst2)
    aA, aB, s20, s21 = _sc_layer2(srcp, dstp, tA, tB, ad2_0, ad2_1)
    return _tc_post(aA, aB, s20, s21, b2)
